# Initial kernel scaffold; baseline (speedup 1.0000x reference)
import jax
import jax.numpy as jnp
from jax.experimental import pallas as pl


def kernel(x, pos, edge_index, edge_attr, batch, protein_embedding, params):
    raise NotImplementedError("write your pallas kernel here")


# speedup vs baseline: 3.0655x; 3.0655x over previous
import functools

import jax
import jax.numpy as jnp
from jax import lax
from jax.experimental import pallas as pl
from jax.experimental.pallas import tpu as pltpu
from jax.experimental.pallas import tpu_sc as plsc

F32 = jnp.float32

N = 10000
E = 320000
HID = 128
DE = 16
NB = 64
PD = 1280

NW = 32
CH = 128
NCH = 79
EPW = NCH * CH
EPAD = NW * EPW
NACC = 10240
DUMP = N
RPT = NACC // 16
ZR = 128

EBLK = 2048
NBLK = 1000

_pc = pl.pallas_call


@functools.lru_cache(maxsize=None)
def _sc_gather_kernel():
  mesh = plsc.VectorSubcoreMesh(core_axis_name="c", subcore_axis_name="s")

  @functools.partial(
      pl.kernel,
      out_type=(
          jax.ShapeDtypeStruct((EPAD, HID), F32),
          jax.ShapeDtypeStruct((EPAD, HID), F32),
      ),
      mesh=mesh,
      scratch_types=[
          pltpu.VMEM((NCH, CH), jnp.int32),
          pltpu.VMEM((NCH, CH), jnp.int32),
          pltpu.VMEM((CH, HID), F32),
          pltpu.VMEM((CH, HID), F32),
          pltpu.SemaphoreType.DMA,
          pltpu.SemaphoreType.DMA,
          pltpu.SemaphoreType.DMA,
          pltpu.SemaphoreType.DMA,
      ],
  )
  def g(pt, qt, ridx3, cidx3, ga, gb, ia_v, ib_v, buf_a, buf_b,
        sem_a, sem_b, sem_oa, sem_ob):
    wid = lax.axis_index("c") * 16 + lax.axis_index("s")
    pltpu.sync_copy(ridx3.at[wid], ia_v)
    pltpu.sync_copy(cidx3.at[wid], ib_v)

    def body(c, _):
      @pl.when(c > 0)
      def _():
        pltpu.make_async_copy(ga.at[pl.ds(0, CH)], buf_a, sem_oa).wait()
        pltpu.make_async_copy(gb.at[pl.ds(0, CH)], buf_b, sem_ob).wait()

      da = pltpu.async_copy(pt.at[ia_v.at[c]], buf_a, sem_a)
      db = pltpu.async_copy(qt.at[ib_v.at[c]], buf_b, sem_b)
      base = wid * EPW + c * CH
      da.wait()
      pltpu.async_copy(buf_a, ga.at[pl.ds(base, CH)], sem_oa)
      db.wait()
      pltpu.async_copy(buf_b, gb.at[pl.ds(base, CH)], sem_ob)
      return ()

    lax.fori_loop(0, NCH, body, ())
    pltpu.make_async_copy(ga.at[pl.ds(0, CH)], buf_a, sem_oa).wait()
    pltpu.make_async_copy(gb.at[pl.ds(0, CH)], buf_b, sem_ob).wait()

  return g


def _sc_gather_call(pt, qt, ridx3, cidx3):
  return _sc_gather_kernel()(pt, qt, ridx3, cidx3)


@functools.lru_cache(maxsize=None)
def _sc_cd_kernel():
  mesh = plsc.VectorSubcoreMesh(core_axis_name="c", subcore_axis_name="s")

  @functools.partial(
      pl.kernel,
      out_type=jax.ShapeDtypeStruct((EPAD, 16), F32),
      mesh=mesh,
      scratch_types=[
          pltpu.VMEM((NCH, CH), jnp.int32),
          pltpu.VMEM((NCH, CH), jnp.int32),
          pltpu.VMEM((CH, 16), F32),
          pltpu.VMEM((CH, 16), F32),
          pltpu.VMEM((CH, 16), F32),
          pltpu.SemaphoreType.DMA,
          pltpu.SemaphoreType.DMA,
      ],
      compiler_params=pltpu.CompilerParams(use_tc_tiling_on_sc=False),
  )
  def g(ct, ridx3, cidx3, cd, ia_v, ib_v, ra, rb, dv, sem_a, sem_b):
    wid = lax.axis_index("c") * 16 + lax.axis_index("s")
    pltpu.sync_copy(ridx3.at[wid], ia_v)
    pltpu.sync_copy(cidx3.at[wid], ib_v)

    def body(c, _):
      da = pltpu.async_copy(ct.at[ia_v.at[c]], ra, sem_a)
      db = pltpu.async_copy(ct.at[ib_v.at[c]], rb, sem_b)
      da.wait()
      db.wait()

      def sub(r, _):
        dv[r, pl.ds(0, 16)] = ra[r, pl.ds(0, 16)] - rb[r, pl.ds(0, 16)]
        return ()

      lax.fori_loop(0, CH, sub, ())
      pltpu.sync_copy(dv, cd.at[pl.ds(wid * EPW + c * CH, CH)])
      return ()

    lax.fori_loop(0, NCH, body, ())

  return g


def _sc_cd_call(ct, ridx3, cidx3):
  return _sc_cd_kernel()(ct, ridx3, cidx3)


def _make_scatter(width, use_tiling):
  mesh = plsc.VectorSubcoreMesh(core_axis_name="c", subcore_axis_name="s")
  params = None if use_tiling else pltpu.CompilerParams(
      use_tc_tiling_on_sc=False)

  @functools.partial(
      pl.kernel,
      out_type=jax.ShapeDtypeStruct((2, NACC, width), F32),
      mesh=mesh,
      scratch_types=[
          pltpu.VMEM((NCH, CH), jnp.int32),
          pltpu.VMEM((CH, width), F32),
          pltpu.VMEM((ZR, width), F32),
          pltpu.VMEM_SHARED((NACC, width), F32),
      ],
      compiler_params=params,
  )
  def s(vals, idx3, out, idx_v, val_v, z_v, acc):
    cc = lax.axis_index("c")
    ss = lax.axis_index("s")
    wid = cc * 16 + ss

    def zb(i, _):
      for j in range(width // 16):
        z_v[i, pl.ds(j * 16, 16)] = jnp.zeros((16,), F32)
      return ()

    lax.fori_loop(0, ZR, zb, ())

    def zs(k, _):
      pltpu.sync_copy(z_v, acc.at[pl.ds(ss * RPT + k * ZR, ZR)])
      return ()

    lax.fori_loop(0, RPT // ZR, zs, ())
    plsc.subcore_barrier()

    pltpu.sync_copy(idx3.at[wid], idx_v)

    def body(c, _):
      pltpu.sync_copy(vals.at[pl.ds(wid * EPW + c * CH, CH)], val_v)
      pltpu.sync_copy(val_v, acc.at[idx_v.at[c]], add=True)
      return ()

    lax.fori_loop(0, NCH, body, ())
    plsc.subcore_barrier()

    def ob(k, _):
      r0 = ss * RPT + k * ZR
      pltpu.sync_copy(acc.at[pl.ds(r0, ZR)], out.at[cc, pl.ds(r0, ZR)])
      return ()

    lax.fori_loop(0, RPT // ZR, ob, ())

  return s


@functools.lru_cache(maxsize=None)
def _sc_scatter_h_kernel():
  return _make_scatter(HID, True)


@functools.lru_cache(maxsize=None)
def _sc_scatter_c_kernel():
  return _make_scatter(16, False)


def _sc_scatter_h_call(ef, idx3):
  return _sc_scatter_h_kernel()(ef, idx3)


def _sc_scatter_c_call(tr, idx3):
  return _sc_scatter_c_kernel()(tr, idx3)


def _full(shape):
  return pl.BlockSpec(shape, lambda i: (0,) * len(shape))


def _init_body(x, eit, bei, w0a, w0b, b0, h_out, pt_out, qt_out):
  h = jnp.dot(x[...], eit[...], preferred_element_type=F32) + bei[...]
  h_out[...] = h
  pt_out[...] = jnp.dot(h, w0a[...], preferred_element_type=F32) + b0[...]
  qt_out[...] = jnp.dot(h, w0b[...], preferred_element_type=F32)


def _tc_init(x, eit, bei, w0a, w0b, b0):
  grid = N // NBLK
  return _pc(
      _init_body,
      grid=(grid,),
      in_specs=[
          pl.BlockSpec((NBLK, HID), lambda i: (i, 0)),
          _full((HID, HID)), _full((1, HID)),
          _full((HID, HID)), _full((HID, HID)), _full((1, HID)),
      ],
      out_specs=[
          pl.BlockSpec((NBLK, HID), lambda i: (i, 0)),
          pl.BlockSpec((NBLK, HID), lambda i: (i, 0)),
          pl.BlockSpec((NBLK, HID), lambda i: (i, 0)),
      ],
      out_shape=[
          jax.ShapeDtypeStruct((N, HID), F32),
          jax.ShapeDtypeStruct((N, HID), F32),
          jax.ShapeDtypeStruct((N, HID), F32),
      ],
  )(x, eit, bei, w0a, w0b, b0)


def _edge_body(ga, gb, cd_ref, ea, wr, w0d, w1, b1, wat, ba, c0, bc0, c1,
               ef_out, tr_out):
  pre = ga[...] + gb[...]
  cd = cd_ref[...]
  radial = jnp.sum(cd * cd, axis=1, keepdims=True)
  norm = jnp.sqrt(radial) + 1e-8
  cdn = cd / norm
  preact = pre + radial * wr[...] + jnp.dot(
      ea[...], w0d[...], preferred_element_type=F32)
  m = jax.nn.silu(preact)
  m = jax.nn.silu(jnp.dot(m, w1[...], preferred_element_type=F32) + b1[...])
  att = jax.nn.sigmoid(
      jnp.dot(m, wat[...], preferred_element_type=F32) + ba[...])
  ef = m * att
  t0 = jax.nn.silu(jnp.dot(ef, c0[...], preferred_element_type=F32) + bc0[...])
  tt = jnp.tanh(jnp.dot(t0, c1[...], preferred_element_type=F32))
  tr = cdn * tt
  lane = lax.broadcasted_iota(jnp.int32, tr.shape, 1)
  tr = jnp.where(lane == 3, 1.0, tr)
  ef_out[...] = ef
  tr_out[...] = tr


def _tc_edge(ga, gb, cd, ea, wr, w0d, w1, b1, wat, ba, c0, bc0, c1):
  grid = EPAD // EBLK
  return _pc(
      _edge_body,
      grid=(grid,),
      in_specs=[
          pl.BlockSpec((EBLK, HID), lambda i: (i, 0)),
          pl.BlockSpec((EBLK, HID), lambda i: (i, 0)),
          pl.BlockSpec((EBLK, 16), lambda i: (i, 0)),
          pl.BlockSpec((EBLK, DE), lambda i: (i, 0)),
          _full((1, HID)), _full((DE, HID)),
          _full((HID, HID)), _full((1, HID)),
          _full((HID, 1)), _full((1, 1)),
          _full((HID, HID)), _full((1, HID)), _full((HID, 1)),
      ],
      out_specs=[
          pl.BlockSpec((EBLK, HID), lambda i: (i, 0)),
          pl.BlockSpec((EBLK, 16), lambda i: (i, 0)),
      ],
      out_shape=[
          jax.ShapeDtypeStruct((EPAD, HID), F32),
          jax.ShapeDtypeStruct((EPAD, 16), F32),
      ],
  )(ga, gb, cd, ea, wr, w0d, w1, b1, wat, ba, c0, bc0, c1)


def _node_common(h_ref, agg2, n0a, n0b, bn0, n1, bn1):
  agg = agg2[0] + agg2[1]
  h = h_ref[...]
  nm = jax.nn.silu(
      jnp.dot(h, n0a[...], preferred_element_type=F32)
      + jnp.dot(agg, n0b[...], preferred_element_type=F32) + bn0[...])
  nm = jnp.dot(nm, n1[...], preferred_element_type=F32) + bn1[...]
  return h + nm


def _node_body(h_ref, coord, aggh2, aggc2, n0a, n0b, bn0, n1, bn1,
               w0a, w0b, b0, h_out, co_out, pt_out, qt_out):
  hn = _node_common(h_ref, aggh2, n0a, n0b, bn0, n1, bn1)
  ac = aggc2[0] + aggc2[1]
  cnt = ac[:, 3:4]
  upd = ac / jnp.maximum(cnt, 1.0)
  lane = lax.broadcasted_iota(jnp.int32, upd.shape, 1)
  cn = coord[...] + jnp.where(lane < 3, upd, 0.0)
  h_out[...] = hn
  co_out[...] = cn
  pt_out[...] = jnp.dot(hn, w0a[...], preferred_element_type=F32) + b0[...]
  qt_out[...] = jnp.dot(hn, w0b[...], preferred_element_type=F32)


def _tc_node(h, coord, aggh2, aggc2, n0a, n0b, bn0, n1, bn1, w0a, w0b, b0):
  grid = N // NBLK
  return _pc(
      _node_body,
      grid=(grid,),
      in_specs=[
          pl.BlockSpec((NBLK, HID), lambda i: (i, 0)),
          pl.BlockSpec((NBLK, 16), lambda i: (i, 0)),
          pl.BlockSpec((2, NBLK, HID), lambda i: (0, i, 0)),
          pl.BlockSpec((2, NBLK, 16), lambda i: (0, i, 0)),
          _full((HID, HID)), _full((HID, HID)), _full((1, HID)),
          _full((HID, HID)), _full((1, HID)),
          _full((HID, HID)), _full((HID, HID)), _full((1, HID)),
      ],
      out_specs=[
          pl.BlockSpec((NBLK, HID), lambda i: (i, 0)),
          pl.BlockSpec((NBLK, 16), lambda i: (i, 0)),
          pl.BlockSpec((NBLK, HID), lambda i: (i, 0)),
          pl.BlockSpec((NBLK, HID), lambda i: (i, 0)),
      ],
      out_shape=[
          jax.ShapeDtypeStruct((N, HID), F32),
          jax.ShapeDtypeStruct((N, 16), F32),
          jax.ShapeDtypeStruct((N, HID), F32),
          jax.ShapeDtypeStruct((N, HID), F32),
      ],
  )(h, coord, aggh2, aggc2, n0a, n0b, bn0, n1, bn1, w0a, w0b, b0)


def _node_last_body(h_ref, aggh2, n0a, n0b, bn0, n1, bn1, h_out):
  h_out[...] = _node_common(h_ref, aggh2, n0a, n0b, bn0, n1, bn1)


def _tc_node_last(h, aggh2, n0a, n0b, bn0, n1, bn1):
  grid = N // NBLK
  return _pc(
      _node_last_body,
      grid=(grid,),
      in_specs=[
          pl.BlockSpec((NBLK, HID), lambda i: (i, 0)),
          pl.BlockSpec((2, NBLK, HID), lambda i: (0, i, 0)),
          _full((HID, HID)), _full((HID, HID)), _full((1, HID)),
          _full((HID, HID)), _full((1, HID)),
      ],
      out_specs=[pl.BlockSpec((NBLK, HID), lambda i: (i, 0))],
      out_shape=[jax.ShapeDtypeStruct((N, HID), F32)],
  )(h, aggh2, n0a, n0b, bn0, n1, bn1)[0]


def _final_body(h_ref, bcol, eot, beo, pe, pt, bp, cat, cbt, bc, c1t, bc1,
                out, sums, cnts):
  i = pl.program_id(0)

  @pl.when(i == 0)
  def _():
    sums[...] = jnp.zeros_like(sums)
    cnts[...] = jnp.zeros_like(cnts)

  ho = jnp.dot(h_ref[...], eot[...], preferred_element_type=F32) + beo[...]
  ids = lax.broadcasted_iota(jnp.int32, (NBLK, NB), 1)
  mask = (bcol[...] == ids).astype(F32)
  dn = (((0,), (0,)), ((), ()))
  sums[...] += lax.dot_general(mask, ho, dn, preferred_element_type=F32)
  cnts[...] += lax.dot_general(
      mask, jnp.ones((NBLK, 1), F32), dn, preferred_element_type=F32)

  @pl.when(i == pl.num_programs(0) - 1)
  def _():
    drug = sums[...] / jnp.maximum(cnts[...], 1.0)
    prot = jax.nn.relu(
        jnp.dot(pe[...], pt[...], preferred_element_type=F32) + bp[...])
    z = jax.nn.relu(
        jnp.dot(drug, cat[...], preferred_element_type=F32)
        + jnp.dot(prot, cbt[...], preferred_element_type=F32) + bc[...])
    out[...] = jnp.dot(z, c1t[...], preferred_element_type=F32) + bc1[...]


def _tc_final(h, bcol, eot, beo, pe, pt, bp, cat, cbt, bc, c1t, bc1):
  grid = N // NBLK
  return _pc(
      _final_body,
      grid=(grid,),
      in_specs=[
          pl.BlockSpec((NBLK, HID), lambda i: (i, 0)),
          pl.BlockSpec((NBLK, 1), lambda i: (i, 0)),
          _full((HID, HID)), _full((1, HID)),
          _full((NB, PD)), _full((PD, 256)), _full((1, 256)),
          _full((HID, 512)), _full((256, 512)), _full((1, 512)),
          _full((512, 1)), _full((1, 1)),
      ],
      out_specs=[pl.BlockSpec((NB, 1), lambda i: (0, 0))],
      out_shape=[jax.ShapeDtypeStruct((NB, 1), F32)],
      scratch_shapes=[pltpu.VMEM((NB, HID), F32), pltpu.VMEM((NB, 1), F32)],
  )(h, bcol, eot, beo, pe, pt, bp, cat, cbt, bc, c1t, bc1)[0]


def _row(b):
  return b.reshape(1, -1)


def kernel(x, pos, edge_index, edge_attr, batch, protein_embedding, params):
  x = x.astype(F32)
  pos16 = jnp.pad(pos.astype(F32), ((0, 0), (0, 13)))
  row = edge_index[0].astype(jnp.int32)
  col = edge_index[1].astype(jnp.int32)
  npad = EPAD - E
  ridx3 = jnp.pad(row, (0, npad)).reshape(NW, NCH, CH)
  cidx3 = jnp.pad(col, (0, npad)).reshape(NW, NCH, CH)
  sidx3 = jnp.pad(row, (0, npad), constant_values=DUMP).reshape(NW, NCH, CH)
  ea = jnp.pad(edge_attr.astype(F32), ((0, npad), (0, 0)))
  bcol = batch.astype(jnp.int32).reshape(N, 1)
  pe = protein_embedding.astype(F32)

  gcl = params["gcl"]

  def sp(l):
    p = gcl[l]
    w0 = p["edge0"]["W"]
    return dict(
        w0a=w0[:, :HID].T, w0b=w0[:, HID:2 * HID].T,
        wr=_row(w0[:, 2 * HID]), w0d=w0[:, 2 * HID + 1:].T,
        b0=_row(p["edge0"]["b"]),
        w1=p["edge1"]["W"].T, b1=_row(p["edge1"]["b"]),
        wat=p["att"]["W"].T, ba=_row(p["att"]["b"]),
        c0=p["coord0"]["W"].T, bc0=_row(p["coord0"]["b"]),
        c1=p["coord1"]["W"].T,
        n0a=p["node0"]["W"][:, :HID].T, n0b=p["node0"]["W"][:, HID:].T,
        bn0=_row(p["node0"]["b"]),
        n1=p["node1"]["W"].T, bn1=_row(p["node1"]["b"]),
    )

  w = [sp(l) for l in range(4)]

  h, ptab, qtab = _tc_init(
      x, params["emb_in"]["W"].T, _row(params["emb_in"]["b"]),
      w[0]["w0a"], w[0]["w0b"], w[0]["b0"])
  coord = pos16

  for l in range(4):
    wl = w[l]
    ga, gb = _sc_gather_call(ptab, qtab, ridx3, cidx3)
    cd = _sc_cd_call(coord, ridx3, cidx3)
    ef, tr = _tc_edge(
        ga, gb, cd, ea, wl["wr"], wl["w0d"], wl["w1"], wl["b1"], wl["wat"],
        wl["ba"], wl["c0"], wl["bc0"], wl["c1"])
    aggh2 = _sc_scatter_h_call(ef, sidx3)
    aggc2 = _sc_scatter_c_call(tr, sidx3)
    if l < 3:
      wn = w[l + 1]
      h, coord, ptab, qtab = _tc_node(
          h, coord, aggh2, aggc2, wl["n0a"], wl["n0b"], wl["bn0"],
          wl["n1"], wl["bn1"], wn["w0a"], wn["w0b"], wn["b0"])
    else:
      h = _tc_node_last(
          h, aggh2, wl["n0a"], wl["n0b"], wl["bn0"], wl["n1"], wl["bn1"])

  cw = params["comb0"]["W"]
  logits = _tc_final(
      h, bcol, params["emb_out"]["W"].T, _row(params["emb_out"]["b"]),
      pe, params["prot"]["W"].T, _row(params["prot"]["b"]),
      cw[:, :HID].T, cw[:, HID:].T, _row(params["comb0"]["b"]),
      params["comb1"]["W"].T, _row(params["comb1"]["b"]))

  return logits[:, 0], jnp.zeros((NB, 1, 1), F32)
